# baseline (speedup 1.0000x reference)
import functools

import jax
import jax.numpy as jnp
import numpy as np
from jax import lax
from jax.experimental import pallas as pl
from jax.experimental.pallas import tpu as pltpu
from jax.experimental.pallas import tpu_sc as plsc

_T = 2048
_F = _T // 2 + 1
_FPAD = 1152
_K = 4
_PMAX = 64
_PMIN = 32
_NP = _PMAX - _PMIN + 1
_FOLDPAD = 2176


def _build_wdft():
    t = np.arange(_T, dtype=np.float64)
    f = np.arange(_F, dtype=np.float64)
    ang = (2.0 * np.pi / _T) * np.outer(t, f)
    wc = np.zeros((_T, _FPAD), dtype=np.float32)
    ws = np.zeros((_T, _FPAD), dtype=np.float32)
    wc[:, :_F] = np.cos(ang).astype(np.float32)
    ws[:, :_F] = -np.sin(ang).astype(np.float32)
    wc[:, 0] = 0.0
    ws[:, 0] = 0.0
    return wc, ws


def _build_wfold() -> np.ndarray:
    w = np.zeros((_T, _FOLDPAD), dtype=np.float32)
    for j in range(_NP):
        p = _PMIN + j
        cycles = _T // p
        start = _T - cycles * p
        tt = np.arange(start, _T)
        w[tt, j * _PMAX + ((tt - start) % p)] = 1.0
    return w


def _build_lut() -> np.ndarray:
    lut = np.zeros((256,), dtype=np.float32)
    for kk in range(128):
        p = int(np.clip(_T // max(kk, 1), _PMIN, _PMAX))
        lut[kk] = (p - _PMIN) * _PMAX
        lut[128 + kk] = 1.0 / (_T // p)
    return lut


_WCOS, _WSIN = _build_wdft()
_WFOLD = _build_wfold().astype(jnp.bfloat16)
_LUT = _build_lut()

_DN = (((1,), (0,)), ((), ()))


def _dft_body(x_ref, wc_ref, ws_ref, m_ref):
    c = jax.lax.dot_general(x_ref[...], wc_ref[...], _DN,
                            preferred_element_type=jnp.float32,
                            precision=jax.lax.Precision.HIGHEST)
    s = jax.lax.dot_general(x_ref[...], ws_ref[...], _DN,
                            preferred_element_type=jnp.float32,
                            precision=jax.lax.Precision.HIGHEST)
    m_ref[...] = c * c + s * s


def _fold_body(x_ref, w_ref, y_ref):
    seqs = x_ref[...]
    xh = seqs.astype(jnp.bfloat16)
    xl = (seqs - xh.astype(jnp.float32)).astype(jnp.bfloat16)
    y_ref[...] = (
        jax.lax.dot_general(xh, w_ref[...], _DN,
                            preferred_element_type=jnp.float32)
        + jax.lax.dot_general(xl, w_ref[...], _DN,
                              preferred_element_type=jnp.float32)
    )


_BN = 1024
_LANES = 16
_NWORK = 32
_ROWS_PER_W = _BN // _NWORK
_NCHUNK = _FPAD // _LANES


def _sc_select(ymag_hbm, yfold_hbm, lut_hbm, out_hbm,
               mag_a, mag_b, o_a, o_b, lut_v,
               sem_a, sem_b, sem_sa, sem_sb, sem_oa, sem_ob):
    wid = lax.axis_index("s") * 2 + lax.axis_index("c")
    base = wid * _ROWS_PER_W
    last = base + _ROWS_PER_W - 1
    iota = lax.broadcasted_iota(jnp.int32, (_LANES,), 0)
    neg1 = jnp.full((_LANES,), -1.0, jnp.float32)
    big = jnp.full((_LANES,), 1 << 30, jnp.int32)

    pltpu.sync_copy(lut_hbm, lut_v)

    def mag_copy(row, buf, sem):
        return pltpu.make_async_copy(
            ymag_hbm.at[pl.ds(pl.multiple_of(row * _FPAD, 8), _FPAD)], buf, sem)

    def out_copy(row, buf, sem):
        return pltpu.make_async_copy(
            buf, out_hbm.at[pl.ds(pl.multiple_of(row * (_K * _PMAX), 8), _K * _PMAX)], sem)

    mag_copy(base, mag_a, sem_a).start()
    mag_copy(base + 1, mag_b, sem_b).start()

    def process(g, row, mag_v, o_v, sem_mag, sem_seg, sem_out):
        mag_copy(row, mag_v, sem_mag).wait()

        def chunk_body(i, carry):
            v1, v2, v3, v4, i1, i2, i3, i4 = carry
            m = mag_v[pl.ds(i * _LANES, _LANES)]
            gi = iota + i * _LANES
            b1 = m > v1
            nv1 = jnp.where(b1, m, v1)
            ni1 = jnp.where(b1, gi, i1)
            m2 = jnp.where(b1, v1, m)
            g2 = jnp.where(b1, i1, gi)
            b2 = m2 > v2
            nv2 = jnp.where(b2, m2, v2)
            ni2 = jnp.where(b2, g2, i2)
            m3 = jnp.where(b2, v2, m2)
            g3 = jnp.where(b2, i2, g2)
            b3 = m3 > v3
            nv3 = jnp.where(b3, m3, v3)
            ni3 = jnp.where(b3, g3, i3)
            m4 = jnp.where(b3, v3, m3)
            g4 = jnp.where(b3, i3, g3)
            b4 = m4 > v4
            nv4 = jnp.where(b4, m4, v4)
            ni4 = jnp.where(b4, g4, i4)
            return nv1, nv2, nv3, nv4, ni1, ni2, ni3, ni4

        init = (neg1, neg1, neg1, neg1, big, big, big, big)
        v1, v2, v3, v4, i1, i2, i3, i4 = lax.fori_loop(
            0, _NCHUNK, chunk_body, init)

        pltpu.make_async_copy(
            ymag_hbm.at[pl.ds(pl.multiple_of(jnp.minimum(row + 2, last) * _FPAD, 8), _FPAD)],
            mag_v, sem_mag).start()

        offs, invcs = [], []
        for _k in range(_K):
            vm = jnp.maximum(jnp.maximum(v1, v2), jnp.maximum(v3, v4))
            maxv = jnp.max(vm)
            maxv_b = jnp.full((_LANES,), maxv, jnp.float32)
            cand = jnp.minimum(
                jnp.minimum(jnp.where(v1 == maxv_b, i1, big),
                            jnp.where(v2 == maxv_b, i2, big)),
                jnp.minimum(jnp.where(v3 == maxv_b, i3, big),
                            jnp.where(v4 == maxv_b, i4, big)))
            gidx = jnp.min(cand)
            gidx_b = jnp.full((_LANES,), gidx, jnp.int32)
            v1 = jnp.where(i1 == gidx_b, neg1, v1)
            v2 = jnp.where(i2 == gidx_b, neg1, v2)
            v3 = jnp.where(i3 == gidx_b, neg1, v3)
            v4 = jnp.where(i4 == gidx_b, neg1, v4)

            kclamp = jnp.minimum(jnp.maximum(gidx, 1), 127)
            kvec = (jnp.full((_LANES,), kclamp, jnp.int32)
                    + jnp.where(iota == 1, 128, 0))
            vals = plsc.load_gather(lut_v, [kvec])
            offs.append(vals[0].astype(jnp.int32))
            invcs.append(jnp.full((_LANES,), vals[1], jnp.float32))

        @pl.when(g > 0)
        def _():
            out_copy(row - 2, o_v, sem_out).wait()

        for k in range(_K):
            pltpu.make_async_copy(
                yfold_hbm.at[pl.ds(pl.multiple_of(row * _FOLDPAD + offs[k], 8), _PMAX)],
                o_v.at[pl.ds(k * _PMAX, _PMAX)], sem_seg).start()
        for k in range(_K):
            pltpu.make_async_copy(
                yfold_hbm.at[pl.ds(pl.multiple_of(row * _FOLDPAD + offs[k], 8), _PMAX)],
                o_v.at[pl.ds(k * _PMAX, _PMAX)], sem_seg).wait()
        for k in range(_K):
            for jj in range(_PMAX // _LANES):
                sl = pl.ds(k * _PMAX + jj * _LANES, _LANES)
                o_v[sl] = o_v[sl] * invcs[k]

        out_copy(row, o_v, sem_out).start()

    def pair_body(g, carry):
        process(g, base + 2 * g, mag_a, o_a, sem_a, sem_sa, sem_oa)
        process(g, base + 2 * g + 1, mag_b, o_b, sem_b, sem_sb, sem_ob)
        return carry

    lax.fori_loop(0, _ROWS_PER_W // 2, pair_body, 0)

    mag_copy(last, mag_a, sem_a).wait()
    mag_copy(last, mag_b, sem_b).wait()
    out_copy(last - 1, o_a, sem_oa).wait()
    out_copy(last, o_b, sem_ob).wait()


@jax.jit
def kernel(x):
    B, T, N = x.shape
    BN = B * N
    seqs = jnp.transpose(x, (0, 2, 1)).reshape(BN, T)
    wcos = jnp.asarray(_WCOS)
    wsin = jnp.asarray(_WSIN)
    wfold = jnp.asarray(_WFOLD)

    rb = 256
    ymag = pl.pallas_call(
        _dft_body,
        grid=(BN // rb,),
        in_specs=[
            pl.BlockSpec((rb, _T), lambda i: (i, 0)),
            pl.BlockSpec((_T, _FPAD), lambda i: (0, 0)),
            pl.BlockSpec((_T, _FPAD), lambda i: (0, 0)),
        ],
        out_specs=pl.BlockSpec((rb, _FPAD), lambda i: (i, 0)),
        out_shape=jax.ShapeDtypeStruct((BN, _FPAD), jnp.float32),
    )(seqs, wcos, wsin)

    yfold = pl.pallas_call(
        _fold_body,
        grid=(BN // rb,),
        in_specs=[
            pl.BlockSpec((rb, _T), lambda i: (i, 0)),
            pl.BlockSpec((_T, _FOLDPAD), lambda i: (0, 0)),
        ],
        out_specs=pl.BlockSpec((rb, _FOLDPAD), lambda i: (i, 0)),
        out_shape=jax.ShapeDtypeStruct((BN, _FOLDPAD), jnp.float32),
    )(seqs, wfold)

    sc_call = functools.partial(
        pl.kernel,
        mesh=plsc.VectorSubcoreMesh(core_axis_name="c", subcore_axis_name="s"),
        compiler_params=pltpu.CompilerParams(needs_layout_passes=False),
        out_type=jax.ShapeDtypeStruct((BN * _K * _PMAX,), jnp.float32),
        scratch_types=[
            pltpu.VMEM((_FPAD,), jnp.float32),
            pltpu.VMEM((_FPAD,), jnp.float32),
            pltpu.VMEM((_K * _PMAX,), jnp.float32),
            pltpu.VMEM((_K * _PMAX,), jnp.float32),
            pltpu.VMEM((256,), jnp.float32),
            pltpu.SemaphoreType.DMA,
            pltpu.SemaphoreType.DMA,
            pltpu.SemaphoreType.DMA,
            pltpu.SemaphoreType.DMA,
            pltpu.SemaphoreType.DMA,
            pltpu.SemaphoreType.DMA,
        ],
    )(_sc_select)
    out = sc_call(ymag.reshape(BN * _FPAD), yfold.reshape(BN * _FOLDPAD),
                  jnp.asarray(_LUT))

    return out.reshape(B, N, _K, _PMAX).transpose(0, 2, 3, 1)


# speedup vs baseline: 1.0050x; 1.0050x over previous
import functools

import jax
import jax.numpy as jnp
import numpy as np
from jax import lax
from jax.experimental import pallas as pl
from jax.experimental.pallas import tpu as pltpu
from jax.experimental.pallas import tpu_sc as plsc

_T = 2048
_F = _T // 2 + 1
_FPAD = 1152
_K = 4
_PMAX = 64
_PMIN = 32
_NP = _PMAX - _PMIN + 1
_FOLDPAD = 2176


def _build_wdft():
    t = np.arange(_T, dtype=np.float64)
    f = np.arange(_F, dtype=np.float64)
    ang = (2.0 * np.pi / _T) * np.outer(t, f)
    wc = np.zeros((_T, _FPAD), dtype=np.float32)
    ws = np.zeros((_T, _FPAD), dtype=np.float32)
    wc[:, :_F] = np.cos(ang).astype(np.float32)
    ws[:, :_F] = -np.sin(ang).astype(np.float32)
    wc[:, 0] = 0.0
    ws[:, 0] = 0.0
    return wc, ws


def _build_wfold() -> np.ndarray:
    w = np.zeros((_T, _FOLDPAD), dtype=np.float32)
    for j in range(_NP):
        p = _PMIN + j
        cycles = _T // p
        start = _T - cycles * p
        tt = np.arange(start, _T)
        w[tt, j * _PMAX + ((tt - start) % p)] = 1.0
    return w


def _build_lut() -> np.ndarray:
    lut = np.zeros((256,), dtype=np.float32)
    for kk in range(128):
        p = int(np.clip(_T // max(kk, 1), _PMIN, _PMAX))
        lut[kk] = (p - _PMIN) * _PMAX
        lut[128 + kk] = 1.0 / (_T // p)
    return lut


_WCOS, _WSIN = _build_wdft()
_WFOLD = _build_wfold().astype(jnp.bfloat16)
_LUT = _build_lut()

_DN = (((1,), (0,)), ((), ()))


def _dft_body(x_ref, wc_ref, ws_ref, m_ref):
    c = jax.lax.dot_general(x_ref[...], wc_ref[...], _DN,
                            preferred_element_type=jnp.float32,
                            precision=jax.lax.Precision.HIGHEST)
    s = jax.lax.dot_general(x_ref[...], ws_ref[...], _DN,
                            preferred_element_type=jnp.float32,
                            precision=jax.lax.Precision.HIGHEST)
    m_ref[...] = c * c + s * s


def _fold_body(x_ref, w_ref, y_ref):
    seqs = x_ref[...]
    xh = seqs.astype(jnp.bfloat16)
    xl = (seqs - xh.astype(jnp.float32)).astype(jnp.bfloat16)
    y_ref[...] = (
        jax.lax.dot_general(xh, w_ref[...], _DN,
                            preferred_element_type=jnp.float32)
        + jax.lax.dot_general(xl, w_ref[...], _DN,
                              preferred_element_type=jnp.float32)
    )


_BN = 1024
_LANES = 16
_NWORK = 32
_ROWS_PER_W = _BN // _NWORK
_NCHUNK = _FPAD // _LANES


def _sc_select(ymag_hbm, yfold_hbm, lut_hbm, out_hbm,
               mag_a, mag_b, o_a, o_b, lut_v,
               sem_a, sem_b, sem_sa, sem_sb, sem_oa, sem_ob):
    wid = lax.axis_index("s") * 2 + lax.axis_index("c")
    base = wid * _ROWS_PER_W
    last = base + _ROWS_PER_W - 1
    iota = lax.broadcasted_iota(jnp.int32, (_LANES,), 0)
    neg1 = jnp.full((_LANES,), -1.0, jnp.float32)
    big = jnp.full((_LANES,), 1 << 30, jnp.int32)

    pltpu.sync_copy(lut_hbm, lut_v)

    def mag_copy(row, buf, sem):
        return pltpu.make_async_copy(
            ymag_hbm.at[pl.ds(pl.multiple_of(row * _FPAD, 8), _FPAD)], buf, sem)

    def out_copy(row, buf, sem):
        return pltpu.make_async_copy(
            buf, out_hbm.at[pl.ds(pl.multiple_of(row * (_K * _PMAX), 8), _K * _PMAX)], sem)

    mag_copy(base, mag_a, sem_a).start()
    mag_copy(base + 1, mag_b, sem_b).start()

    def process(g, row, mag_v, o_v, sem_mag, sem_seg, sem_out):
        mag_copy(row, mag_v, sem_mag).wait()

        def chunk_body(i, carry):
            v1, v2, v3, v4, i1, i2, i3, i4 = carry
            m = mag_v[pl.ds(i * _LANES, _LANES)]
            gi = iota + i * _LANES
            b1 = m > v1
            nv1 = jnp.where(b1, m, v1)
            ni1 = jnp.where(b1, gi, i1)
            m2 = jnp.where(b1, v1, m)
            g2 = jnp.where(b1, i1, gi)
            b2 = m2 > v2
            nv2 = jnp.where(b2, m2, v2)
            ni2 = jnp.where(b2, g2, i2)
            m3 = jnp.where(b2, v2, m2)
            g3 = jnp.where(b2, i2, g2)
            b3 = m3 > v3
            nv3 = jnp.where(b3, m3, v3)
            ni3 = jnp.where(b3, g3, i3)
            m4 = jnp.where(b3, v3, m3)
            g4 = jnp.where(b3, i3, g3)
            b4 = m4 > v4
            nv4 = jnp.where(b4, m4, v4)
            ni4 = jnp.where(b4, g4, i4)
            return nv1, nv2, nv3, nv4, ni1, ni2, ni3, ni4

        init = (neg1, neg1, neg1, neg1, big, big, big, big)
        v1, v2, v3, v4, i1, i2, i3, i4 = lax.fori_loop(
            0, _NCHUNK, chunk_body, init, unroll=8)

        pltpu.make_async_copy(
            ymag_hbm.at[pl.ds(pl.multiple_of(jnp.minimum(row + 2, last) * _FPAD, 8), _FPAD)],
            mag_v, sem_mag).start()

        offs, invcs = [], []
        for _k in range(_K):
            vm = jnp.maximum(jnp.maximum(v1, v2), jnp.maximum(v3, v4))
            maxv = jnp.max(vm)
            maxv_b = jnp.full((_LANES,), maxv, jnp.float32)
            cand = jnp.minimum(
                jnp.minimum(jnp.where(v1 == maxv_b, i1, big),
                            jnp.where(v2 == maxv_b, i2, big)),
                jnp.minimum(jnp.where(v3 == maxv_b, i3, big),
                            jnp.where(v4 == maxv_b, i4, big)))
            gidx = jnp.min(cand)
            gidx_b = jnp.full((_LANES,), gidx, jnp.int32)
            v1 = jnp.where(i1 == gidx_b, neg1, v1)
            v2 = jnp.where(i2 == gidx_b, neg1, v2)
            v3 = jnp.where(i3 == gidx_b, neg1, v3)
            v4 = jnp.where(i4 == gidx_b, neg1, v4)

            kclamp = jnp.minimum(jnp.maximum(gidx, 1), 127)
            kvec = (jnp.full((_LANES,), kclamp, jnp.int32)
                    + jnp.where(iota == 1, 128, 0))
            vals = plsc.load_gather(lut_v, [kvec])
            offs.append(vals[0].astype(jnp.int32))
            invcs.append(jnp.full((_LANES,), vals[1], jnp.float32))

        @pl.when(g > 0)
        def _():
            out_copy(row - 2, o_v, sem_out).wait()

        for k in range(_K):
            pltpu.make_async_copy(
                yfold_hbm.at[pl.ds(pl.multiple_of(row * _FOLDPAD + offs[k], 8), _PMAX)],
                o_v.at[pl.ds(k * _PMAX, _PMAX)], sem_seg).start()
        for k in range(_K):
            pltpu.make_async_copy(
                yfold_hbm.at[pl.ds(pl.multiple_of(row * _FOLDPAD + offs[k], 8), _PMAX)],
                o_v.at[pl.ds(k * _PMAX, _PMAX)], sem_seg).wait()
        for k in range(_K):
            for jj in range(_PMAX // _LANES):
                sl = pl.ds(k * _PMAX + jj * _LANES, _LANES)
                o_v[sl] = o_v[sl] * invcs[k]

        out_copy(row, o_v, sem_out).start()

    def pair_body(g, carry):
        process(g, base + 2 * g, mag_a, o_a, sem_a, sem_sa, sem_oa)
        process(g, base + 2 * g + 1, mag_b, o_b, sem_b, sem_sb, sem_ob)
        return carry

    lax.fori_loop(0, _ROWS_PER_W // 2, pair_body, 0)

    mag_copy(last, mag_a, sem_a).wait()
    mag_copy(last, mag_b, sem_b).wait()
    out_copy(last - 1, o_a, sem_oa).wait()
    out_copy(last, o_b, sem_ob).wait()


@jax.jit
def kernel(x):
    B, T, N = x.shape
    BN = B * N
    seqs = jnp.transpose(x, (0, 2, 1)).reshape(BN, T)
    wcos = jnp.asarray(_WCOS)
    wsin = jnp.asarray(_WSIN)
    wfold = jnp.asarray(_WFOLD)

    rb = 256
    ymag = pl.pallas_call(
        _dft_body,
        grid=(BN // rb,),
        in_specs=[
            pl.BlockSpec((rb, _T), lambda i: (i, 0)),
            pl.BlockSpec((_T, _FPAD), lambda i: (0, 0)),
            pl.BlockSpec((_T, _FPAD), lambda i: (0, 0)),
        ],
        out_specs=pl.BlockSpec((rb, _FPAD), lambda i: (i, 0)),
        out_shape=jax.ShapeDtypeStruct((BN, _FPAD), jnp.float32),
    )(seqs, wcos, wsin)

    yfold = pl.pallas_call(
        _fold_body,
        grid=(BN // rb,),
        in_specs=[
            pl.BlockSpec((rb, _T), lambda i: (i, 0)),
            pl.BlockSpec((_T, _FOLDPAD), lambda i: (0, 0)),
        ],
        out_specs=pl.BlockSpec((rb, _FOLDPAD), lambda i: (i, 0)),
        out_shape=jax.ShapeDtypeStruct((BN, _FOLDPAD), jnp.float32),
    )(seqs, wfold)

    sc_call = functools.partial(
        pl.kernel,
        mesh=plsc.VectorSubcoreMesh(core_axis_name="c", subcore_axis_name="s"),
        compiler_params=pltpu.CompilerParams(needs_layout_passes=False),
        out_type=jax.ShapeDtypeStruct((BN * _K * _PMAX,), jnp.float32),
        scratch_types=[
            pltpu.VMEM((_FPAD,), jnp.float32),
            pltpu.VMEM((_FPAD,), jnp.float32),
            pltpu.VMEM((_K * _PMAX,), jnp.float32),
            pltpu.VMEM((_K * _PMAX,), jnp.float32),
            pltpu.VMEM((256,), jnp.float32),
            pltpu.SemaphoreType.DMA,
            pltpu.SemaphoreType.DMA,
            pltpu.SemaphoreType.DMA,
            pltpu.SemaphoreType.DMA,
            pltpu.SemaphoreType.DMA,
            pltpu.SemaphoreType.DMA,
        ],
    )(_sc_select)
    out = sc_call(ymag.reshape(BN * _FPAD), yfold.reshape(BN * _FOLDPAD),
                  jnp.asarray(_LUT))

    return out.reshape(B, N, _K, _PMAX).transpose(0, 2, 3, 1)
